# baseline (speedup 1.0000x reference)
import functools

import jax
import jax.numpy as jnp
from jax import lax
from jax.experimental import pallas as pl
from jax.experimental.pallas import tpu as pltpu
from jax.experimental.pallas import tpu_sc as plsc

_NULL_ROW = 1000000
_HIDDEN = 64
_BATCH = 16384

_info = plsc.get_sparse_core_info()
_NC, _NS = _info.num_cores, _info.num_subcores
_NW = _NC * _NS
_BPW = _BATCH // _NW

_mesh = plsc.VectorSubcoreMesh(core_axis_name="c", subcore_axis_name="s")


@functools.partial(
    pl.kernel,
    mesh=_mesh,
    out_type=jax.ShapeDtypeStruct((_NW, _BPW * _HIDDEN), jnp.float32),
    scratch_types=[
        pltpu.VMEM((_BPW,), jnp.int32),
        pltpu.VMEM((_BPW,), jnp.int32),
        pltpu.VMEM((_BPW * _HIDDEN,), jnp.float32),
        pltpu.SemaphoreType.DMA,
    ],
)
def _lookup(lbl_hbm, fd_hbm, table_hbm, out_hbm, lbl_v, fd_v, rows_v, sem):
    wid = lax.axis_index("s") * _NC + lax.axis_index("c")
    pltpu.sync_copy(lbl_hbm.at[wid], lbl_v)
    pltpu.sync_copy(fd_hbm.at[wid], fd_v)
    for i in range(_BPW // 16):
        sl = pl.ds(i * 16, 16)
        lbl_v[sl] = jnp.where(fd_v[sl] == 1, _NULL_ROW, lbl_v[sl]) * _HIDDEN

    def issue(g, carry):
        vec = lbl_v[pl.ds(g * 16, 16)]
        for j in range(16):
            pltpu.async_copy(
                table_hbm.at[pl.ds(pl.multiple_of(vec[j], _HIDDEN), _HIDDEN)],
                rows_v.at[pl.ds((g * 16 + j) * _HIDDEN, _HIDDEN)],
                sem,
            )
        return carry

    lax.fori_loop(0, _BPW // 16, issue, 0)
    pltpu.make_async_copy(
        table_hbm.at[pl.ds(0, _BPW * _HIDDEN)], rows_v, sem
    ).wait()
    pltpu.sync_copy(rows_v, out_hbm.at[wid])


def kernel(labels, train, force_drop_ids, embedding_table):
    del train
    lbl2 = labels.reshape(_NW, _BPW).astype(jnp.int32)
    fd2 = force_drop_ids.reshape(_NW, _BPW).astype(jnp.int32)
    out = _lookup(lbl2, fd2, embedding_table.reshape(-1))
    return out.reshape(_BATCH, _HIDDEN)


# speedup vs baseline: 1.0002x; 1.0002x over previous
import functools

import jax
import jax.numpy as jnp
from jax import lax
from jax.experimental import pallas as pl
from jax.experimental.pallas import tpu as pltpu
from jax.experimental.pallas import tpu_sc as plsc

_NULL_ROW = 1000000
_HIDDEN = 64
_BATCH = 16384
_NSTREAM = 8
_CHUNK = 64

_info = plsc.get_sparse_core_info()
_NC, _NS = _info.num_cores, _info.num_subcores
_NW = _NC * _NS
_BPW = _BATCH // _NW

_mesh = plsc.VectorSubcoreMesh(core_axis_name="c", subcore_axis_name="s")


@functools.partial(
    pl.kernel,
    mesh=_mesh,
    out_type=jax.ShapeDtypeStruct((_NW, _BPW, _HIDDEN), jnp.float32),
    scratch_types=[
        pltpu.VMEM((_NSTREAM, _CHUNK), jnp.int32),
        pltpu.VMEM((_NSTREAM, _CHUNK), jnp.int32),
        pltpu.VMEM((_BPW, _HIDDEN), jnp.float32),
        pltpu.SemaphoreType.DMA,
    ],
    compiler_params=pltpu.CompilerParams(use_tc_tiling_on_sc=False),
)
def _lookup(lbl_hbm, fd_hbm, table_hbm, out_hbm, lbl_v, fd_v, rows_v, sem):
    wid = lax.axis_index("s") * _NC + lax.axis_index("c")
    pltpu.sync_copy(lbl_hbm.at[wid], lbl_v)
    pltpu.sync_copy(fd_hbm.at[wid], fd_v)
    for j in range(_NSTREAM):
        for i in range(_CHUNK // 16):
            sl = (j, pl.ds(i * 16, 16))
            lbl_v[sl] = jnp.where(fd_v[sl] == 1, _NULL_ROW, lbl_v[sl])
    copies = [
        pltpu.async_copy(
            table_hbm.at[lbl_v.at[j]],
            rows_v.at[pl.ds(j * _CHUNK, _CHUNK)],
            sem,
        )
        for j in range(_NSTREAM)
    ]
    for c in copies:
        c.wait()
    pltpu.sync_copy(rows_v, out_hbm.at[wid])


def kernel(labels, train, force_drop_ids, embedding_table):
    del train
    lbl2 = labels.reshape(_NW, _NSTREAM, _CHUNK).astype(jnp.int32)
    fd2 = force_drop_ids.reshape(_NW, _NSTREAM, _CHUNK).astype(jnp.int32)
    out = _lookup(lbl2, fd2, embedding_table)
    return out.reshape(_BATCH, _HIDDEN)


# speedup vs baseline: 3.0356x; 3.0350x over previous
import functools

import jax
import jax.numpy as jnp
from jax import lax
from jax.experimental import pallas as pl
from jax.experimental.pallas import tpu as pltpu
from jax.experimental.pallas import tpu_sc as plsc

_NULL_ROW = 1000000
_HIDDEN = 64
_BATCH = 16384
_ROWS = _NULL_ROW + 1
_RC = 16384
_NCH = 61
_TAIL0 = _NCH * _RC
_TAILN = _ROWS - _TAIL0
_NB = _NCH + 1
_LPT = 1024

_info = plsc.get_sparse_core_info()
_NC, _NS = _info.num_cores, _info.num_subcores

_mesh = plsc.VectorSubcoreMesh(core_axis_name="c", subcore_axis_name="s")


@functools.partial(
    pl.kernel,
    mesh=_mesh,
    out_type=jax.ShapeDtypeStruct((_HIDDEN, _BATCH), jnp.float32),
    scratch_types=[
        pltpu.VMEM((_LPT,), jnp.int32),
        pltpu.VMEM((_LPT,), jnp.int32),
        pltpu.VMEM((_LPT + 16,), jnp.int32),
        pltpu.VMEM((32, _LPT), jnp.float32),
        pltpu.VMEM((128,), jnp.int32),
        pltpu.VMEM((128,), jnp.float32),
        pltpu.VMEM((_TAILN, _HIDDEN), jnp.float32),
        pltpu.SMEM((_NB,), jnp.int32),
        pltpu.SMEM((_NB,), jnp.int32),
        pltpu.SMEM((_NB,), jnp.int32),
        pltpu.VMEM_SHARED((8 * _RC,), jnp.float32),
        pltpu.VMEM_SHARED((8 * _RC,), jnp.float32),
        pltpu.SemaphoreType.DMA,
        pltpu.SemaphoreType.DMA,
        pltpu.SemaphoreType.DMA,
        pltpu.SemaphoreType.DMA,
    ],
    compiler_params=pltpu.CompilerParams(needs_layout_passes=False),
)
def _lookup(
    lbl_hbm,
    fd_hbm,
    tbl_hbm,
    tail_hbm,
    out_hbm,
    lbl_v,
    fd_v,
    packed_l,
    outs_v,
    idx_v,
    rows_v,
    tail_v,
    cnt_s,
    pfx_s,
    cur_s,
    slab0,
    slab1,
    sem0,
    sem1,
    semg,
    semt,
):
    s = lax.axis_index("c")
    t = lax.axis_index("s")

    pltpu.sync_copy(lbl_hbm.at[t], lbl_v)
    pltpu.sync_copy(fd_hbm.at[t], fd_v)
    pltpu.async_copy(tail_hbm, tail_v, semt)

    lane = lax.iota(jnp.int32, 16)

    def chunk_of(g):
        sl = pl.ds(g * 16, 16)
        lbl = jnp.where(fd_v[sl] == 1, _NULL_ROW, lbl_v[sl])
        tail_m = lbl >= _TAIL0
        ch = jnp.where(tail_m, _NCH, lax.shift_right_logical(lbl, 14))
        off = jnp.where(tail_m, lbl - _TAIL0, lbl & (_RC - 1))
        return ch, off

    for c in range(_NB):
        cnt_s[c] = 0

    def count_g(g, carry):
        ch, _ = chunk_of(g)
        for c in range(_NB):
            npc = plsc.all_reduce_population_count(ch == c)
            cnt_s[c] = cnt_s[c] + npc[0]
        return carry

    lax.fori_loop(0, _LPT // 16, count_g, 0)

    run = 0
    for c in range(_NB):
        n = cnt_s[c]
        pfx_s[c] = run
        cur_s[c] = run
        run = run + n

    def place_g(g, carry):
        ch, off = chunk_of(g)
        packed = off | lax.shift_left(lane + g * 16, 14)
        for c in range(_NB):
            m = ch == c
            mi = m.astype(jnp.int32)
            dst = (plsc.cumsum(mi) - mi) + cur_s[c]
            plsc.store_scatter(packed_l, [dst], packed, mask=m)
            npc = plsc.all_reduce_population_count(m)
            cur_s[c] = cur_s[c] + npc[0]
        return carry

    lax.fori_loop(0, _LPT // 16, place_g, 0)

    def issue_slab(hg_i, c, slab, sem):
        for h in range(8):
            row = (s * 4 + hg_i) * 8 + h
            pltpu.async_copy(
                tbl_hbm.at[row, pl.ds(c * _RC, _RC)],
                slab.at[pl.ds(h * _RC, _RC)],
                sem,
            )

    def wait_slab(slab, sem):
        row0 = s * 0
        for h in range(8):
            pltpu.make_async_copy(
                tbl_hbm.at[row0, pl.ds(0, _RC)],
                slab.at[pl.ds(h * _RC, _RC)],
                sem,
            ).wait()

    def gather_chunk(hg_i, c, slab):
        n_c = cnt_s[c]
        base = pfx_s[c]
        ng = lax.shift_right_logical(n_c + 15, 4)

        def grp(k, carry):
            v = packed_l[pl.ds(base + k * 16, 16)]
            off = v & (_RC - 1)
            pos = lax.shift_right_logical(v, 14)
            m = (lane + k * 16) < n_c
            for h in range(8):
                idx_v[pl.ds(h * 16, 16)] = off + h * _RC
            pltpu.async_copy(slab.at[idx_v], rows_v, semg).wait()
            for h in range(8):
                hl = jnp.full((16,), hg_i * 8 + h, jnp.int32)
                vals = rows_v[pl.ds(h * 16, 16)]
                plsc.store_scatter(outs_v, [hl, pos], vals, mask=m)
            return carry

        lax.fori_loop(0, ng, grp, 0)

    for hg_i in range(4):

        @pl.when(t == 0)
        def _():
            issue_slab(hg_i, 0, slab0, sem0)

        def stage(c, carry):
            @pl.when(t == 0)
            def _():
                @pl.when(lax.rem(c, 2) == 0)
                def _():
                    issue_slab(hg_i, c + 1, slab1, sem1)
                    wait_slab(slab0, sem0)

                @pl.when(lax.rem(c, 2) == 1)
                def _():
                    issue_slab(hg_i, c + 1, slab0, sem0)
                    wait_slab(slab1, sem1)

            plsc.subcore_barrier()

            @pl.when(lax.rem(c, 2) == 0)
            def _():
                gather_chunk(hg_i, c, slab0)

            @pl.when(lax.rem(c, 2) == 1)
            def _():
                gather_chunk(hg_i, c, slab1)

            plsc.subcore_barrier()
            return carry

        lax.fori_loop(0, _NCH - 1, stage, 0)

        @pl.when(t == 0)
        def _():
            wait_slab(slab0, sem0)

        plsc.subcore_barrier()
        gather_chunk(hg_i, _NCH - 1, slab0)
        plsc.subcore_barrier()

    pltpu.make_async_copy(tail_hbm, tail_v, semt).wait()
    n_t = cnt_s[_NCH]
    base_t = pfx_s[_NCH]
    ng_t = lax.shift_right_logical(n_t + 15, 4)

    def tgrp(k, carry):
        v = packed_l[pl.ds(base_t + k * 16, 16)]
        off = v & (_RC - 1)
        pos = lax.shift_right_logical(v, 14)
        m = (lane + k * 16) < n_t
        for hl in range(32):
            hv = jnp.full((16,), hl, jnp.int32)
            vals = plsc.load_gather(tail_v, [off, hv + s * 32], mask=m)
            plsc.store_scatter(outs_v, [hv, pos], vals, mask=m)
        return carry

    lax.fori_loop(0, ng_t, tgrp, 0)

    pltpu.sync_copy(
        outs_v,
        out_hbm.at[pl.ds(s * 32, 32), pl.ds(t * _LPT, _LPT)],
    )


def kernel(labels, train, force_drop_ids, embedding_table):
    del train
    lbl2 = labels.reshape(_NS, _LPT).astype(jnp.int32)
    fd2 = force_drop_ids.reshape(_NS, _LPT).astype(jnp.int32)
    out_t = _lookup(
        lbl2, fd2, embedding_table.T, embedding_table[_TAIL0:]
    )
    return out_t.T


# speedup vs baseline: 3.0771x; 1.0137x over previous
import functools

import jax
import jax.numpy as jnp
from jax import lax
from jax.experimental import pallas as pl
from jax.experimental.pallas import tpu as pltpu
from jax.experimental.pallas import tpu_sc as plsc

_NULL_ROW = 1000000
_HIDDEN = 64
_BATCH = 16384
_ROWS = _NULL_ROW + 1
_RC = 16384
_NCH = 61
_TAIL0 = _NCH * _RC
_TAILN = _ROWS - _TAIL0
_NB = _NCH + 1
_LPT = 1024

_info = plsc.get_sparse_core_info()
_NC, _NS = _info.num_cores, _info.num_subcores

_mesh = plsc.VectorSubcoreMesh(core_axis_name="c", subcore_axis_name="s")


@functools.partial(
    pl.kernel,
    mesh=_mesh,
    out_type=jax.ShapeDtypeStruct((_HIDDEN, _BATCH), jnp.float32),
    scratch_types=[
        pltpu.VMEM((_LPT,), jnp.int32),
        pltpu.VMEM((_LPT,), jnp.int32),
        pltpu.VMEM((_LPT + 16,), jnp.int32),
        pltpu.VMEM((32, _LPT), jnp.float32),
        pltpu.VMEM((128,), jnp.int32),
        pltpu.VMEM((128,), jnp.float32),
        pltpu.VMEM((64,), jnp.int32),
        pltpu.VMEM((_TAILN, _HIDDEN), jnp.float32),
        pltpu.SMEM((_NB,), jnp.int32),
        pltpu.SMEM((_NB,), jnp.int32),
        pltpu.SMEM((_NB,), jnp.int32),
        pltpu.VMEM_SHARED((8 * _RC,), jnp.float32),
        pltpu.VMEM_SHARED((8 * _RC,), jnp.float32),
        pltpu.SemaphoreType.DMA,
        pltpu.SemaphoreType.DMA,
        pltpu.SemaphoreType.DMA,
        pltpu.SemaphoreType.DMA,
    ],
    compiler_params=pltpu.CompilerParams(needs_layout_passes=False),
)
def _lookup(
    lbl_hbm,
    fd_hbm,
    tbl_hbm,
    tail_hbm,
    out_hbm,
    lbl_v,
    fd_v,
    packed_l,
    outs_v,
    idx_v,
    rows_v,
    counts_v,
    tail_v,
    cnt_s,
    pfx_s,
    cur_s,
    slab0,
    slab1,
    sem0,
    sem1,
    semg,
    semt,
):
    s = lax.axis_index("c")
    t = lax.axis_index("s")

    pltpu.sync_copy(lbl_hbm.at[t], lbl_v)
    pltpu.sync_copy(fd_hbm.at[t], fd_v)
    pltpu.async_copy(tail_hbm, tail_v, semt)

    lane = lax.iota(jnp.int32, 16)

    def chunk_of(g):
        sl = pl.ds(g * 16, 16)
        lbl = jnp.where(fd_v[sl] == 1, _NULL_ROW, lbl_v[sl])
        tail_m = lbl >= _TAIL0
        ch = jnp.where(tail_m, _NCH, lax.shift_right_logical(lbl, 14))
        off = jnp.where(tail_m, lbl - _TAIL0, lbl & (_RC - 1))
        return ch, off

    zero16 = jnp.zeros((16,), jnp.int32)
    one16 = jnp.ones((16,), jnp.int32)
    for cg in range(4):
        counts_v[pl.ds(cg * 16, 16)] = zero16

    def count_g(g, carry):
        ch, _ = chunk_of(g)
        plsc.addupdate_scatter(counts_v, [ch], one16)
        return carry

    lax.fori_loop(0, _LPT // 16, count_g, 0)

    for cg in range(4):
        v = counts_v[pl.ds(cg * 16, 16)]
        for j in range(16):
            if cg * 16 + j < _NB:
                cnt_s[cg * 16 + j] = v[j]

    run = 0
    for c in range(_NB):
        n = cnt_s[c]
        pfx_s[c] = run
        cur_s[c] = run
        run = run + n

    def place_g(g, carry):
        ch, off = chunk_of(g)
        packed = off | lax.shift_left(lane + g * 16, 14)
        for c in range(_NB):
            m = ch == c
            mi = m.astype(jnp.int32)
            dst = (plsc.cumsum(mi) - mi) + cur_s[c]
            plsc.store_scatter(packed_l, [dst], packed, mask=m)
            npc = plsc.all_reduce_population_count(m)
            cur_s[c] = cur_s[c] + npc[0]
        return carry

    lax.fori_loop(0, _LPT // 16, place_g, 0)

    def issue_slab(hg_i, c, slab, sem):
        for h in range(8):
            row = (s * 4 + hg_i) * 8 + h
            pltpu.async_copy(
                tbl_hbm.at[row, pl.ds(c * _RC, _RC)],
                slab.at[pl.ds(h * _RC, _RC)],
                sem,
            )

    def wait_slab(slab, sem):
        row0 = s * 0
        for h in range(8):
            pltpu.make_async_copy(
                tbl_hbm.at[row0, pl.ds(0, _RC)],
                slab.at[pl.ds(h * _RC, _RC)],
                sem,
            ).wait()

    def gather_chunk(hg_i, c, slab):
        n_c = cnt_s[c]
        base = pfx_s[c]
        ng = lax.shift_right_logical(n_c + 15, 4)

        def grp(k, carry):
            v = packed_l[pl.ds(base + k * 16, 16)]
            off = v & (_RC - 1)
            pos = lax.shift_right_logical(v, 14)
            m = (lane + k * 16) < n_c
            for h in range(8):
                idx_v[pl.ds(h * 16, 16)] = off + h * _RC
            pltpu.async_copy(slab.at[idx_v], rows_v, semg).wait()
            for h in range(8):
                hl = jnp.full((16,), hg_i * 8 + h, jnp.int32)
                vals = rows_v[pl.ds(h * 16, 16)]
                plsc.store_scatter(outs_v, [hl, pos], vals, mask=m)
            return carry

        lax.fori_loop(0, ng, grp, 0)

    for hg_i in range(4):

        @pl.when(t == 0)
        def _():
            issue_slab(hg_i, 0, slab0, sem0)

        def stage(c, carry):
            @pl.when(t == 0)
            def _():
                @pl.when(lax.rem(c, 2) == 0)
                def _():
                    issue_slab(hg_i, c + 1, slab1, sem1)
                    wait_slab(slab0, sem0)

                @pl.when(lax.rem(c, 2) == 1)
                def _():
                    issue_slab(hg_i, c + 1, slab0, sem0)
                    wait_slab(slab1, sem1)

            plsc.subcore_barrier()

            @pl.when(lax.rem(c, 2) == 0)
            def _():
                gather_chunk(hg_i, c, slab0)

            @pl.when(lax.rem(c, 2) == 1)
            def _():
                gather_chunk(hg_i, c, slab1)

            plsc.subcore_barrier()
            return carry

        lax.fori_loop(0, _NCH - 1, stage, 0)

        @pl.when(t == 0)
        def _():
            wait_slab(slab0, sem0)

        plsc.subcore_barrier()
        gather_chunk(hg_i, _NCH - 1, slab0)
        plsc.subcore_barrier()

    pltpu.make_async_copy(tail_hbm, tail_v, semt).wait()
    n_t = cnt_s[_NCH]
    base_t = pfx_s[_NCH]
    ng_t = lax.shift_right_logical(n_t + 15, 4)

    def tgrp(k, carry):
        v = packed_l[pl.ds(base_t + k * 16, 16)]
        off = v & (_RC - 1)
        pos = lax.shift_right_logical(v, 14)
        m = (lane + k * 16) < n_t
        for hl in range(32):
            hv = jnp.full((16,), hl, jnp.int32)
            vals = plsc.load_gather(tail_v, [off, hv + s * 32], mask=m)
            plsc.store_scatter(outs_v, [hv, pos], vals, mask=m)
        return carry

    lax.fori_loop(0, ng_t, tgrp, 0)

    pltpu.sync_copy(
        outs_v,
        out_hbm.at[pl.ds(s * 32, 32), pl.ds(t * _LPT, _LPT)],
    )


def kernel(labels, train, force_drop_ids, embedding_table):
    del train
    lbl2 = labels.reshape(_NS, _LPT).astype(jnp.int32)
    fd2 = force_drop_ids.reshape(_NS, _LPT).astype(jnp.int32)
    out_t = _lookup(
        lbl2, fd2, embedding_table.T, embedding_table[_TAIL0:]
    )
    return out_t.T
